# Initial kernel scaffold; baseline (speedup 1.0000x reference)
import jax
import jax.numpy as jnp
from jax.experimental import pallas as pl


def kernel(features, rois, W1, b1, W2, b2, W3, b3, Wc1, bc1, Wc2, bc2, Wr1, br1, Wr2, br2, Wcls, bcls, Wbox, bbox):
    raise NotImplementedError("write your pallas kernel here")


# speedup vs baseline: 12.2777x; 12.2777x over previous
import jax
import jax.numpy as jnp
from jax import lax
from jax.experimental import pallas as pl
from jax.experimental.pallas import tpu as pltpu

F32 = jnp.float32
SCALE = 1.0 / 16.0
B = 16
KPAD = 1024
NCHUNK = KPAD // B


def _compose_body(w1v_ref, w2t_ref, w3t_ref, b1_ref, b2_ref, b3_ref,
                  wt_ref, beff_ref, s21_ref):
    s21_ref[...] = jnp.zeros((6400, 256), F32)
    w1v = w1v_ref[...]
    for d in range(3):
        for e in range(3):
            res = jnp.dot(w1v, w2t_ref[3 * d + e],
                          preferred_element_type=F32)
            for bb in range(3):
                start = ((bb + e) * 5 + d) * 256
                s21_ref[pl.ds(start, 768), :] = (
                    s21_ref[pl.ds(start, 768), :] + res[bb * 768:(bb + 1) * 768, :])
    wt_ref[...] = jnp.zeros((16384, 256), F32)
    s21 = s21_ref[...]
    for f in range(3):
        for g in range(3):
            res = jnp.dot(s21, w3t_ref[3 * f + g],
                          preferred_element_type=F32)
            for vv in range(5):
                start = ((vv + g) * 8 + f) * 256
                wt_ref[pl.ds(start, 1280), :] = (
                    wt_ref[pl.ds(start, 1280), :] + res[vv * 1280:(vv + 1) * 1280, :])
    w2sum = w2t_ref[0]
    w3sum = w3t_ref[0]
    for i in range(1, 9):
        w2sum = w2sum + w2t_ref[i]
        w3sum = w3sum + w3t_ref[i]
    s2 = jnp.dot(b1_ref[...], w2sum, preferred_element_type=F32) + b2_ref[...]
    beff_ref[...] = jnp.dot(s2, w3sum, preferred_element_type=F32) + b3_ref[...]


def _interp_matrix(lo, binsz, shift, limit, ncols):
    pyf = lax.broadcasted_iota(jnp.int32, (B, 8, 1), 1).astype(F32)
    idx = lax.broadcasted_iota(jnp.int32, (B, 8, ncols), 2).astype(F32)
    lo3 = lo[:, :, None]
    bin3 = binsz[:, :, None]
    sh3 = shift[:, :, None]
    acc = jnp.zeros((B, 8, ncols), F32)
    for s_off in (0.25, 0.75):
        pos = jnp.clip(lo3 + (pyf + s_off) * bin3, 0.0, limit)
        p0 = jnp.floor(pos)
        lw = pos - p0
        r0 = p0 + sh3
        r1 = jnp.minimum(p0 + 1.0, limit) + sh3
        acc = (acc
               + jnp.where(idx == r0, 1.0 - lw, 0.0)
               + jnp.where(idx == r1, lw, 0.0))
    return acc * 0.5


def _main_body(rois_ref, fcat_ref, wt_ref, beff_ref,
               wc1_ref, bc1_ref, wc2_ref, bc2_ref,
               wr1_ref, br1_ref, wr2_ref, br2_ref,
               wcls_ref, bcls_ref, wbox_ref, bbox_ref,
               cls_ref, box_ref, e_ref):
    r = rois_ref[...]
    bcol = r[:, 0:1]
    x1 = r[:, 1:2] * SCALE
    y1 = r[:, 2:3] * SCALE
    x2 = r[:, 3:4] * SCALE
    y2 = r[:, 4:5] * SCALE
    bin_w = jnp.maximum(x2 - x1, 1.0) / 7.0
    bin_h = jnp.maximum(y2 - y1, 1.0) / 7.0

    ay = _interp_matrix(y1, bin_h, 50.0 * bcol, 49.0, 100)
    ax = _interp_matrix(x1, bin_w, jnp.zeros_like(bcol), 49.0, 50)

    rmat = jnp.dot(ay.reshape(B * 8, 100), fcat_ref[...],
                   preferred_element_type=F32)
    r4 = rmat.reshape(B, 8, 50, 256)

    terms = []
    for px in range(7):
        w = ax[:, px:px + 1, :][:, :, :, None]
        terms.append(jnp.sum(r4 * w, axis=2))
    terms.append(jnp.zeros((B, 8, 256), F32))
    pooled = jnp.stack(terms, axis=1)

    e = jnp.dot(pooled.reshape(B, 16384), wt_ref[...],
                preferred_element_type=F32) + beff_ref[0:1, :]

    mc = jnp.maximum(jnp.dot(e, wc1_ref[...], preferred_element_type=F32)
                     + bc1_ref[0:1, :], 0.0)
    mc = jnp.maximum(jnp.dot(mc, wc2_ref[...], preferred_element_type=F32)
                     + bc2_ref[0:1, :], 0.0)
    mr = jnp.maximum(jnp.dot(e, wr1_ref[...], preferred_element_type=F32)
                     + br1_ref[0:1, :], 0.0)
    mr = jnp.maximum(jnp.dot(mr, wr2_ref[...], preferred_element_type=F32)
                     + br2_ref[0:1, :], 0.0)
    cls_ref[...] = (jnp.dot(mc, wcls_ref[...], preferred_element_type=F32)
                    + bcls_ref[0:1, :])
    box_ref[...] = (jnp.dot(mr, wbox_ref[...], preferred_element_type=F32)
                    + bbox_ref[0:1, :])
    e_ref[...] = e


def _full(shape):
    nd = len(shape)
    return pl.BlockSpec(shape, lambda i, _n=nd: (0,) * _n)


def _row8(b):
    return jnp.broadcast_to(b[None, :], (8, b.shape[0])).astype(F32)


def kernel(features, rois, W1, b1, W2, b2, W3, b3, Wc1, bc1, Wc2, bc2,
           Wr1, br1, Wr2, br2, Wcls, bcls, Wbox, bbox):
    C = 256
    fcat = jnp.transpose(features, (0, 2, 3, 1)).reshape(100, 50 * C)
    w1v = jnp.transpose(W1, (3, 2, 1, 0)).reshape(9 * C, C)
    w2t = jnp.transpose(W2, (2, 3, 1, 0)).reshape(9, C, C)
    w3t = jnp.transpose(W3, (2, 3, 1, 0)).reshape(9, C, C)

    wt, beff = pl.pallas_call(
        _compose_body,
        out_shape=[jax.ShapeDtypeStruct((16384, C), F32),
                   jax.ShapeDtypeStruct((8, C), F32)],
        scratch_shapes=[pltpu.VMEM((6400, C), F32)],
    )(w1v, w2t, w3t, _row8(b1), _row8(b2), _row8(b3))

    k = rois.shape[0]
    rois_pad = jnp.zeros((KPAD, 8), F32).at[:k, :5].set(rois)

    grid_specs = dict(
        grid=(NCHUNK,),
        in_specs=[
            pl.BlockSpec((B, 8), lambda i: (i, 0)),
            _full((100, 50 * C)),
            _full((16384, C)),
            _full((8, C)),
            _full((C, 512)), _full((8, 512)),
            _full((512, 512)), _full((8, 512)),
            _full((C, 512)), _full((8, 512)),
            _full((512, 512)), _full((8, 512)),
            _full((512, 81)), _full((8, 81)),
            _full((512, 324)), _full((8, 324)),
        ],
        out_specs=[
            pl.BlockSpec((B, 81), lambda i: (i, 0)),
            pl.BlockSpec((B, 324), lambda i: (i, 0)),
            pl.BlockSpec((B, C), lambda i: (i, 0)),
        ],
    )
    cls_p, box_p, e_p = pl.pallas_call(
        _main_body,
        out_shape=[jax.ShapeDtypeStruct((KPAD, 81), F32),
                   jax.ShapeDtypeStruct((KPAD, 324), F32),
                   jax.ShapeDtypeStruct((KPAD, C), F32)],
        **grid_specs,
    )(rois_pad, fcat, wt, beff,
      Wc1.T, _row8(bc1), Wc2.T, _row8(bc2),
      Wr1.T, _row8(br1), Wr2.T, _row8(br2),
      Wcls.T, _row8(bcls), Wbox.T, _row8(bbox))

    return (cls_p[:k], box_p[:k], e_p[:k].reshape(k, C, 1, 1))


# speedup vs baseline: 20.3571x; 1.6580x over previous
import jax
import jax.numpy as jnp
from jax import lax
from jax.experimental import pallas as pl
from jax.experimental.pallas import tpu as pltpu

F32 = jnp.float32
SCALE = 1.0 / 16.0
B = 16
KPAD = 1024
NCHUNK = KPAD // B


def _compose_body(w1v_ref, w2t_ref, w3t_ref, b1_ref, b2_ref, b3_ref,
                  wt_ref, beff_ref, s21_ref):
    s21_ref[...] = jnp.zeros((6400, 256), F32)
    w1v = w1v_ref[...]
    for d in range(3):
        for e in range(3):
            res = jnp.dot(w1v, w2t_ref[3 * d + e],
                          preferred_element_type=F32)
            for bb in range(3):
                start = ((bb + e) * 5 + d) * 256
                s21_ref[pl.ds(start, 768), :] = (
                    s21_ref[pl.ds(start, 768), :] + res[bb * 768:(bb + 1) * 768, :])
    wt_ref[...] = jnp.zeros((16384, 256), F32)
    s21 = s21_ref[...]
    for f in range(3):
        for g in range(3):
            res = jnp.dot(s21, w3t_ref[3 * f + g],
                          preferred_element_type=F32)
            for vv in range(5):
                start = ((vv + g) * 8 + f) * 256
                wt_ref[pl.ds(start, 1280), :] = (
                    wt_ref[pl.ds(start, 1280), :] + res[vv * 1280:(vv + 1) * 1280, :])
    w2sum = w2t_ref[0]
    w3sum = w3t_ref[0]
    for i in range(1, 9):
        w2sum = w2sum + w2t_ref[i]
        w3sum = w3sum + w3t_ref[i]
    s2 = jnp.dot(b1_ref[...], w2sum, preferred_element_type=F32) + b2_ref[...]
    beff_ref[...] = jnp.dot(s2, w3sum, preferred_element_type=F32) + b3_ref[...]


def _interp_matrix(lo, binsz, shift, limit, ncols):
    pyf = lax.broadcasted_iota(jnp.int32, (B, 8, 1), 1).astype(F32)
    idx = lax.broadcasted_iota(jnp.int32, (B, 8, ncols), 2).astype(F32)
    lo3 = lo[:, :, None]
    bin3 = binsz[:, :, None]
    sh3 = shift[:, :, None]
    acc = jnp.zeros((B, 8, ncols), F32)
    for s_off in (0.25, 0.75):
        pos = jnp.clip(lo3 + (pyf + s_off) * bin3, 0.0, limit)
        p0 = jnp.floor(pos)
        lw = pos - p0
        r0 = p0 + sh3
        r1 = jnp.minimum(p0 + 1.0, limit) + sh3
        acc = (acc
               + jnp.where(idx == r0, 1.0 - lw, 0.0)
               + jnp.where(idx == r1, lw, 0.0))
    return acc * 0.5


def _main_body(rois_ref, fcat_ref, wt_ref, beff_ref, e_ref):
    r = rois_ref[...]
    bcol = r[:, 0:1]
    x1 = r[:, 1:2] * SCALE
    y1 = r[:, 2:3] * SCALE
    x2 = r[:, 3:4] * SCALE
    y2 = r[:, 4:5] * SCALE
    bin_w = jnp.maximum(x2 - x1, 1.0) / 7.0
    bin_h = jnp.maximum(y2 - y1, 1.0) / 7.0

    ay = _interp_matrix(y1, bin_h, 50.0 * bcol, 49.0, 100)
    ax = _interp_matrix(x1, bin_w, jnp.zeros_like(bcol), 49.0, 50)

    rmat = jnp.dot(ay.reshape(B * 8, 100), fcat_ref[...],
                   preferred_element_type=F32)
    r4 = rmat.reshape(B, 8, 50, 256)
    rt = jnp.transpose(r4, (0, 2, 1, 3)).reshape(B, 50, 2048)

    pooled = lax.dot_general(ax, rt, (((2,), (1,)), ((0,), (0,))),
                             preferred_element_type=F32)

    e_ref[...] = jnp.dot(pooled.reshape(B, 16384), wt_ref[...],
                         preferred_element_type=F32) + beff_ref[0:1, :]


def _head_body(e_ref, wc1_ref, bc1_ref, wc2_ref, bc2_ref,
               wr1_ref, br1_ref, wr2_ref, br2_ref,
               wcls_ref, bcls_ref, wbox_ref, bbox_ref,
               cls_ref, box_ref):
    e = e_ref[...]
    mc = jnp.maximum(jnp.dot(e, wc1_ref[...], preferred_element_type=F32)
                     + bc1_ref[0:1, :], 0.0)
    mc = jnp.maximum(jnp.dot(mc, wc2_ref[...], preferred_element_type=F32)
                     + bc2_ref[0:1, :], 0.0)
    mr = jnp.maximum(jnp.dot(e, wr1_ref[...], preferred_element_type=F32)
                     + br1_ref[0:1, :], 0.0)
    mr = jnp.maximum(jnp.dot(mr, wr2_ref[...], preferred_element_type=F32)
                     + br2_ref[0:1, :], 0.0)
    cls_ref[...] = (jnp.dot(mc, wcls_ref[...], preferred_element_type=F32)
                    + bcls_ref[0:1, :])
    box_ref[...] = (jnp.dot(mr, wbox_ref[...], preferred_element_type=F32)
                    + bbox_ref[0:1, :])


def _full(shape):
    nd = len(shape)
    return pl.BlockSpec(shape, lambda i, _n=nd: (0,) * _n)


def _row8(b):
    return jnp.broadcast_to(b[None, :], (8, b.shape[0])).astype(F32)


def kernel(features, rois, W1, b1, W2, b2, W3, b3, Wc1, bc1, Wc2, bc2,
           Wr1, br1, Wr2, br2, Wcls, bcls, Wbox, bbox):
    C = 256
    fcat = jnp.transpose(features, (0, 2, 3, 1)).reshape(100, 50 * C)
    w1v = jnp.transpose(W1, (3, 2, 1, 0)).reshape(9 * C, C)
    w2t = jnp.transpose(W2, (2, 3, 1, 0)).reshape(9, C, C)
    w3t = jnp.transpose(W3, (2, 3, 1, 0)).reshape(9, C, C)

    wt, beff = pl.pallas_call(
        _compose_body,
        out_shape=[jax.ShapeDtypeStruct((16384, C), F32),
                   jax.ShapeDtypeStruct((8, C), F32)],
        scratch_shapes=[pltpu.VMEM((6400, C), F32)],
    )(w1v, w2t, w3t, _row8(b1), _row8(b2), _row8(b3))

    k = rois.shape[0]
    rois_pad = jnp.zeros((KPAD, 8), F32).at[:k, :5].set(rois)

    e_p = pl.pallas_call(
        _main_body,
        grid=(NCHUNK,),
        in_specs=[
            pl.BlockSpec((B, 8), lambda i: (i, 0)),
            _full((100, 50 * C)),
            _full((16384, C)),
            _full((8, C)),
        ],
        out_specs=pl.BlockSpec((B, C), lambda i: (i, 0)),
        out_shape=jax.ShapeDtypeStruct((KPAD, C), F32),
        compiler_params=pltpu.CompilerParams(
            dimension_semantics=("arbitrary",)),
    )(rois_pad, fcat, wt, beff)

    cls_p, box_p = pl.pallas_call(
        _head_body,
        out_shape=[jax.ShapeDtypeStruct((KPAD, 81), F32),
                   jax.ShapeDtypeStruct((KPAD, 324), F32)],
    )(e_p, Wc1.T, _row8(bc1), Wc2.T, _row8(bc2),
      Wr1.T, _row8(br1), Wr2.T, _row8(br2),
      Wcls.T, _row8(bcls), Wbox.T, _row8(bbox))

    return (cls_p[:k], box_p[:k], e_p[:k].reshape(k, C, 1, 1))


# speedup vs baseline: 22.4283x; 1.1017x over previous
import jax
import jax.numpy as jnp
from jax import lax
from jax.experimental import pallas as pl
from jax.experimental.pallas import tpu as pltpu

F32 = jnp.float32
SCALE = 1.0 / 16.0
B = 32
KPAD = 1024
NCHUNK = KPAD // B


def _compose_body(w1v_ref, w2t_ref, w3t_ref, b1_ref, b2_ref, b3_ref,
                  wt_ref, beff_ref, s21_ref):
    s21_ref[...] = jnp.zeros((6400, 256), F32)
    w1v = w1v_ref[...]
    for d in range(3):
        for e in range(3):
            res = jnp.dot(w1v, w2t_ref[3 * d + e],
                          preferred_element_type=F32)
            for bb in range(3):
                start = ((bb + e) * 5 + d) * 256
                s21_ref[pl.ds(start, 768), :] = (
                    s21_ref[pl.ds(start, 768), :] + res[bb * 768:(bb + 1) * 768, :])
    wt_ref[...] = jnp.zeros((16384, 256), F32)
    s21 = s21_ref[...]
    for f in range(3):
        for g in range(3):
            res = jnp.dot(s21, w3t_ref[3 * f + g],
                          preferred_element_type=F32)
            for vv in range(5):
                start = ((vv + g) * 8 + f) * 256
                wt_ref[pl.ds(start, 1280), :] = (
                    wt_ref[pl.ds(start, 1280), :] + res[vv * 1280:(vv + 1) * 1280, :])
    w2sum = w2t_ref[0]
    w3sum = w3t_ref[0]
    for i in range(1, 9):
        w2sum = w2sum + w2t_ref[i]
        w3sum = w3sum + w3t_ref[i]
    s2 = jnp.dot(b1_ref[...], w2sum, preferred_element_type=F32) + b2_ref[...]
    beff_ref[...] = jnp.dot(s2, w3sum, preferred_element_type=F32) + b3_ref[...]


def _interp_matrix(lo, binsz, shift, limit, ncols):
    pyf = lax.broadcasted_iota(jnp.int32, (B, 8, 1), 1).astype(F32)
    idx = lax.broadcasted_iota(jnp.int32, (B, 8, ncols), 2).astype(F32)
    lo3 = lo[:, :, None]
    bin3 = binsz[:, :, None]
    sh3 = shift[:, :, None]
    acc = jnp.zeros((B, 8, ncols), F32)
    for s_off in (0.25, 0.75):
        pos = jnp.clip(lo3 + (pyf + s_off) * bin3, 0.0, limit)
        p0 = jnp.floor(pos)
        lw = pos - p0
        r0 = p0 + sh3
        r1 = jnp.minimum(p0 + 1.0, limit) + sh3
        acc = (acc
               + jnp.where(idx == r0, 1.0 - lw, 0.0)
               + jnp.where(idx == r1, lw, 0.0))
    return acc * 0.5


def _main_body(rois_ref, fcat_ref, wt_ref, beff_ref, e_ref):
    r = rois_ref[...]
    bcol = r[:, 0:1]
    x1 = r[:, 1:2] * SCALE
    y1 = r[:, 2:3] * SCALE
    x2 = r[:, 3:4] * SCALE
    y2 = r[:, 4:5] * SCALE
    bin_w = jnp.maximum(x2 - x1, 1.0) / 7.0
    bin_h = jnp.maximum(y2 - y1, 1.0) / 7.0

    ay = _interp_matrix(y1, bin_h, 50.0 * bcol, 49.0, 100)
    ax = _interp_matrix(x1, bin_w, jnp.zeros_like(bcol), 49.0, 50)

    rmat = jnp.dot(ay.reshape(B * 8, 100), fcat_ref[...],
                   preferred_element_type=F32)
    r4 = rmat.reshape(B, 8, 50, 256)

    pooled = lax.dot_general(ax, r4, (((2,), (2,)), ((0,), (0,))),
                             preferred_element_type=F32)

    e_ref[...] = jnp.dot(pooled.reshape(B, 16384), wt_ref[...],
                         preferred_element_type=F32) + beff_ref[0:1, :]


def _head_body(e_ref, wc1_ref, bc1_ref, wc2_ref, bc2_ref,
               wr1_ref, br1_ref, wr2_ref, br2_ref,
               wcls_ref, bcls_ref, wbox_ref, bbox_ref,
               cls_ref, box_ref):
    e = e_ref[...]
    mc = jnp.maximum(jnp.dot(e, wc1_ref[...], preferred_element_type=F32)
                     + bc1_ref[0:1, :], 0.0)
    mc = jnp.maximum(jnp.dot(mc, wc2_ref[...], preferred_element_type=F32)
                     + bc2_ref[0:1, :], 0.0)
    mr = jnp.maximum(jnp.dot(e, wr1_ref[...], preferred_element_type=F32)
                     + br1_ref[0:1, :], 0.0)
    mr = jnp.maximum(jnp.dot(mr, wr2_ref[...], preferred_element_type=F32)
                     + br2_ref[0:1, :], 0.0)
    cls_ref[...] = (jnp.dot(mc, wcls_ref[...], preferred_element_type=F32)
                    + bcls_ref[0:1, :])
    box_ref[...] = (jnp.dot(mr, wbox_ref[...], preferred_element_type=F32)
                    + bbox_ref[0:1, :])


def _full(shape):
    nd = len(shape)
    return pl.BlockSpec(shape, lambda i, _n=nd: (0,) * _n)


def _row8(b):
    return jnp.broadcast_to(b[None, :], (8, b.shape[0])).astype(F32)


def kernel(features, rois, W1, b1, W2, b2, W3, b3, Wc1, bc1, Wc2, bc2,
           Wr1, br1, Wr2, br2, Wcls, bcls, Wbox, bbox):
    C = 256
    fcat = jnp.transpose(features, (0, 2, 3, 1)).reshape(100, 50 * C)
    w1v = jnp.transpose(W1, (3, 2, 1, 0)).reshape(9 * C, C)
    w2t = jnp.transpose(W2, (2, 3, 1, 0)).reshape(9, C, C)
    w3t = jnp.transpose(W3, (2, 3, 1, 0)).reshape(9, C, C)

    wt, beff = pl.pallas_call(
        _compose_body,
        out_shape=[jax.ShapeDtypeStruct((16384, C), F32),
                   jax.ShapeDtypeStruct((8, C), F32)],
        scratch_shapes=[pltpu.VMEM((6400, C), F32)],
    )(w1v, w2t, w3t, _row8(b1), _row8(b2), _row8(b3))

    k = rois.shape[0]
    rois_pad = jnp.zeros((KPAD, 8), F32).at[:k, :5].set(rois)

    e_p = pl.pallas_call(
        _main_body,
        grid=(NCHUNK,),
        in_specs=[
            pl.BlockSpec((B, 8), lambda i: (i, 0)),
            _full((100, 50 * C)),
            _full((16384, C)),
            _full((8, C)),
        ],
        out_specs=pl.BlockSpec((B, C), lambda i: (i, 0)),
        out_shape=jax.ShapeDtypeStruct((KPAD, C), F32),
        compiler_params=pltpu.CompilerParams(
            dimension_semantics=("arbitrary",)),
    )(rois_pad, fcat, wt, beff)

    cls_p, box_p = pl.pallas_call(
        _head_body,
        out_shape=[jax.ShapeDtypeStruct((KPAD, 81), F32),
                   jax.ShapeDtypeStruct((KPAD, 324), F32)],
    )(e_p, Wc1.T, _row8(bc1), Wc2.T, _row8(bc2),
      Wr1.T, _row8(br1), Wr2.T, _row8(br2),
      Wcls.T, _row8(bcls), Wbox.T, _row8(bbox))

    return (cls_p[:k], box_p[:k], e_p[:k].reshape(k, C, 1, 1))


# speedup vs baseline: 23.1265x; 1.0311x over previous
import jax
import jax.numpy as jnp
from jax import lax
from jax.experimental import pallas as pl
from jax.experimental.pallas import tpu as pltpu

F32 = jnp.float32
SCALE = 1.0 / 16.0
B = 32
KPAD = 1024
NCHUNK = KPAD // B


def _compose_body(w1v_ref, w2t_ref, w3t_ref, b1_ref, b2_ref, b3_ref,
                  wt_ref, beff_ref, s21_ref):
    s21_ref[...] = jnp.zeros((6400, 256), F32)
    w1v = w1v_ref[...]
    for d in range(3):
        for e in range(3):
            res = jnp.dot(w1v, w2t_ref[3 * d + e],
                          preferred_element_type=F32)
            for bb in range(3):
                start = ((bb + e) * 5 + d) * 256
                s21_ref[pl.ds(start, 768), :] = (
                    s21_ref[pl.ds(start, 768), :] + res[bb * 768:(bb + 1) * 768, :])
    wt_ref[...] = jnp.zeros((16384, 256), F32)
    s21 = s21_ref[...]
    for f in range(3):
        for g in range(3):
            res = jnp.dot(s21, w3t_ref[3 * f + g],
                          preferred_element_type=F32)
            for vv in range(5):
                start = ((vv + g) * 8 + f) * 256
                wt_ref[pl.ds(start, 1280), :] = (
                    wt_ref[pl.ds(start, 1280), :] + res[vv * 1280:(vv + 1) * 1280, :])
    w2sum = w2t_ref[0]
    w3sum = w3t_ref[0]
    for i in range(1, 9):
        w2sum = w2sum + w2t_ref[i]
        w3sum = w3sum + w3t_ref[i]
    s2 = jnp.dot(b1_ref[...], w2sum, preferred_element_type=F32) + b2_ref[...]
    beff_ref[...] = jnp.dot(s2, w3sum, preferred_element_type=F32) + b3_ref[...]


def _interp_matrix(lo, binsz, shift, limit, ncols):
    pyf = lax.broadcasted_iota(jnp.int32, (B, 8, 1), 1).astype(F32)
    idx = lax.broadcasted_iota(jnp.int32, (B, 8, ncols), 2).astype(F32)
    lo3 = lo[:, :, None]
    bin3 = binsz[:, :, None]
    sh3 = shift[:, :, None]
    acc = jnp.zeros((B, 8, ncols), F32)
    for s_off in (0.25, 0.75):
        pos = jnp.clip(lo3 + (pyf + s_off) * bin3, 0.0, limit)
        p0 = jnp.floor(pos)
        lw = pos - p0
        r0 = p0 + sh3
        r1 = jnp.minimum(p0 + 1.0, limit) + sh3
        acc = (acc
               + jnp.where(idx == r0, 1.0 - lw, 0.0)
               + jnp.where(idx == r1, lw, 0.0))
    return acc * 0.5


def _main_body(rois_ref, fcat_ref, wt_ref, beff_ref, e_ref):
    r = rois_ref[...]
    bcol = r[:, 0:1]
    x1 = r[:, 1:2] * SCALE
    y1 = r[:, 2:3] * SCALE
    x2 = r[:, 3:4] * SCALE
    y2 = r[:, 4:5] * SCALE
    bin_w = jnp.maximum(x2 - x1, 1.0) / 7.0
    bin_h = jnp.maximum(y2 - y1, 1.0) / 7.0

    ay = _interp_matrix(y1, bin_h, 50.0 * bcol, 49.0, 100)
    ax = _interp_matrix(x1, bin_w, jnp.zeros_like(bcol), 49.0, 50)

    rmat = jnp.dot(ay.reshape(B * 8, 100), fcat_ref[...],
                   preferred_element_type=F32)
    r4 = rmat.reshape(B, 8, 50, 256)

    pooled = lax.dot_general(ax, r4, (((2,), (2,)), ((0,), (0,))),
                             preferred_element_type=F32)

    e_ref[...] = jnp.dot(pooled.reshape(B, 16384), wt_ref[...],
                         preferred_element_type=F32) + beff_ref[0:1, :]


def _head_body(e_ref, wc1_ref, bc1_ref, wc2_ref, bc2_ref,
               wr1_ref, br1_ref, wr2_ref, br2_ref,
               wcls_ref, bcls_ref, wbox_ref, bbox_ref,
               cls_ref, box_ref):
    def dott(x, w_ref):
        return lax.dot_general(x, w_ref[...], (((1,), (1,)), ((), ())),
                               preferred_element_type=F32)

    e = e_ref[...]
    mc = jnp.maximum(dott(e, wc1_ref) + bc1_ref[0:1, :], 0.0)
    mc = jnp.maximum(dott(mc, wc2_ref) + bc2_ref[0:1, :], 0.0)
    mr = jnp.maximum(dott(e, wr1_ref) + br1_ref[0:1, :], 0.0)
    mr = jnp.maximum(dott(mr, wr2_ref) + br2_ref[0:1, :], 0.0)
    cls_ref[...] = dott(mc, wcls_ref) + bcls_ref[0:1, :]
    box_ref[...] = dott(mr, wbox_ref) + bbox_ref[0:1, :]


def _full(shape):
    nd = len(shape)
    return pl.BlockSpec(shape, lambda i, _n=nd: (0,) * _n)


def _row8(b):
    return jnp.broadcast_to(b[None, :], (8, b.shape[0])).astype(F32)


def kernel(features, rois, W1, b1, W2, b2, W3, b3, Wc1, bc1, Wc2, bc2,
           Wr1, br1, Wr2, br2, Wcls, bcls, Wbox, bbox):
    C = 256
    fcat = jnp.transpose(features, (0, 2, 3, 1)).reshape(100, 50 * C)
    w1v = jnp.transpose(W1, (3, 2, 1, 0)).reshape(9 * C, C)
    w2t = jnp.transpose(W2, (2, 3, 1, 0)).reshape(9, C, C)
    w3t = jnp.transpose(W3, (2, 3, 1, 0)).reshape(9, C, C)

    wt, beff = pl.pallas_call(
        _compose_body,
        out_shape=[jax.ShapeDtypeStruct((16384, C), F32),
                   jax.ShapeDtypeStruct((8, C), F32)],
        scratch_shapes=[pltpu.VMEM((6400, C), F32)],
    )(w1v, w2t, w3t, _row8(b1), _row8(b2), _row8(b3))

    k = rois.shape[0]
    rois_pad = jnp.zeros((KPAD, 8), F32).at[:k, :5].set(rois)

    e_p = pl.pallas_call(
        _main_body,
        grid=(NCHUNK,),
        in_specs=[
            pl.BlockSpec((B, 8), lambda i: (i, 0)),
            _full((100, 50 * C)),
            _full((16384, C)),
            _full((8, C)),
        ],
        out_specs=pl.BlockSpec((B, C), lambda i: (i, 0)),
        out_shape=jax.ShapeDtypeStruct((KPAD, C), F32),
        compiler_params=pltpu.CompilerParams(
            dimension_semantics=("arbitrary",)),
    )(rois_pad, fcat, wt, beff)

    cls_p, box_p = pl.pallas_call(
        _head_body,
        out_shape=[jax.ShapeDtypeStruct((KPAD, 81), F32),
                   jax.ShapeDtypeStruct((KPAD, 324), F32)],
    )(e_p, Wc1, _row8(bc1), Wc2, _row8(bc2),
      Wr1, _row8(br1), Wr2, _row8(br2),
      Wcls, _row8(bcls), Wbox, _row8(bbox))

    return (cls_p[:k], box_p[:k], e_p[:k].reshape(k, C, 1, 1))


# speedup vs baseline: 28.0706x; 1.2138x over previous
import jax
import jax.numpy as jnp
from jax import lax
from jax.experimental import pallas as pl
from jax.experimental.pallas import tpu as pltpu

F32 = jnp.float32
SCALE = 1.0 / 16.0
B = 32
KPAD = 1024
NCHUNK = KPAD // B


def _compose_body(w1v_ref, w2t_ref, w3t_ref, b1_ref, b2_ref, b3_ref,
                  wt_ref, beff_ref, s21_ref):
    s21_ref[...] = jnp.zeros((6400, 256), F32)
    w1v = w1v_ref[...]
    for d in range(3):
        for e in range(3):
            res = jnp.dot(w1v, w2t_ref[3 * d + e],
                          preferred_element_type=F32)
            for bb in range(3):
                start = ((bb + e) * 5 + d) * 256
                s21_ref[pl.ds(start, 768), :] = (
                    s21_ref[pl.ds(start, 768), :] + res[bb * 768:(bb + 1) * 768, :])
    wt_ref[...] = jnp.zeros((16384, 256), F32)
    s21 = s21_ref[...]
    for f in range(3):
        for g in range(3):
            res = jnp.dot(s21, w3t_ref[3 * f + g],
                          preferred_element_type=F32)
            for vv in range(5):
                start = ((vv + g) * 8 + f) * 256
                wt_ref[pl.ds(start, 1280), :] = (
                    wt_ref[pl.ds(start, 1280), :] + res[vv * 1280:(vv + 1) * 1280, :])
    w2sum = w2t_ref[0]
    w3sum = w3t_ref[0]
    for i in range(1, 9):
        w2sum = w2sum + w2t_ref[i]
        w3sum = w3sum + w3t_ref[i]
    s2 = jnp.dot(b1_ref[...], w2sum, preferred_element_type=F32) + b2_ref[...]
    beff_ref[...] = jnp.dot(s2, w3sum, preferred_element_type=F32) + b3_ref[...]


def _interp_matrix(lo, binsz, shift, limit, ncols):
    pyf = lax.broadcasted_iota(jnp.int32, (B, 8, 1), 1).astype(F32)
    idx = lax.broadcasted_iota(jnp.int32, (B, 8, ncols), 2).astype(F32)
    lo3 = lo[:, :, None]
    bin3 = binsz[:, :, None]
    sh3 = shift[:, :, None]
    acc = jnp.zeros((B, 8, ncols), F32)
    for s_off in (0.25, 0.75):
        pos = jnp.clip(lo3 + (pyf + s_off) * bin3, 0.0, limit)
        p0 = jnp.floor(pos)
        lw = pos - p0
        r0 = p0 + sh3
        r1 = jnp.minimum(p0 + 1.0, limit) + sh3
        acc = (acc
               + jnp.where(idx == r0, 1.0 - lw, 0.0)
               + jnp.where(idx == r1, lw, 0.0))
    return acc * 0.5


def _main_body(rois_ref, fcat_ref, wt_ref, beff_ref, e_ref, pool_ref):
    r = rois_ref[...]
    bcol = r[:, 0:1]
    x1 = r[:, 1:2] * SCALE
    y1 = r[:, 2:3] * SCALE
    x2 = r[:, 3:4] * SCALE
    y2 = r[:, 4:5] * SCALE
    bin_w = jnp.maximum(x2 - x1, 1.0) / 7.0
    bin_h = jnp.maximum(y2 - y1, 1.0) / 7.0

    ay = _interp_matrix(y1, bin_h, 50.0 * bcol, 49.0, 100)
    ax = _interp_matrix(x1, bin_w, jnp.zeros_like(bcol), 49.0, 50)

    rmat = jnp.dot(ay.reshape(B * 8, 100), fcat_ref[...],
                   preferred_element_type=F32)
    r4 = rmat.reshape(B, 8, 50, 256)

    pooled = lax.dot_general(ax, r4, (((2,), (2,)), ((0,), (0,))),
                             preferred_element_type=F32)

    j = pl.program_id(1)
    pool_ref[pl.ds(j * B, B), :] = pooled.reshape(B, 16384)

    @pl.when(j == 3)
    def _():
        e_ref[...] = jnp.dot(pool_ref[...], wt_ref[...],
                             preferred_element_type=F32) + beff_ref[0:1, :]


def _head_body(e_ref, wc1_ref, bc1_ref, wc2_ref, bc2_ref,
               wr1_ref, br1_ref, wr2_ref, br2_ref,
               wcls_ref, bcls_ref, wbox_ref, bbox_ref,
               cls_ref, box_ref):
    def dott(x, w_ref):
        return lax.dot_general(x, w_ref[...], (((1,), (1,)), ((), ())),
                               preferred_element_type=F32)

    e = e_ref[...]
    mc = jnp.maximum(dott(e, wc1_ref) + bc1_ref[0:1, :], 0.0)
    mc = jnp.maximum(dott(mc, wc2_ref) + bc2_ref[0:1, :], 0.0)
    mr = jnp.maximum(dott(e, wr1_ref) + br1_ref[0:1, :], 0.0)
    mr = jnp.maximum(dott(mr, wr2_ref) + br2_ref[0:1, :], 0.0)
    cls_ref[...] = dott(mc, wcls_ref) + bcls_ref[0:1, :]
    box_ref[...] = dott(mr, wbox_ref) + bbox_ref[0:1, :]


def _full(shape):
    nd = len(shape)
    return pl.BlockSpec(shape, lambda i, _n=nd: (0,) * _n)


def _row8(b):
    return jnp.broadcast_to(b[None, :], (8, b.shape[0])).astype(F32)


def kernel(features, rois, W1, b1, W2, b2, W3, b3, Wc1, bc1, Wc2, bc2,
           Wr1, br1, Wr2, br2, Wcls, bcls, Wbox, bbox):
    C = 256
    fcat = jnp.transpose(features, (0, 2, 3, 1)).reshape(100, 50 * C)
    w1v = jnp.transpose(W1, (3, 2, 1, 0)).reshape(9 * C, C)
    w2t = jnp.transpose(W2, (2, 3, 1, 0)).reshape(9, C, C)
    w3t = jnp.transpose(W3, (2, 3, 1, 0)).reshape(9, C, C)

    wt, beff = pl.pallas_call(
        _compose_body,
        out_shape=[jax.ShapeDtypeStruct((16384, C), F32),
                   jax.ShapeDtypeStruct((8, C), F32)],
        scratch_shapes=[pltpu.VMEM((6400, C), F32)],
    )(w1v, w2t, w3t, _row8(b1), _row8(b2), _row8(b3))

    k = rois.shape[0]
    rois_pad = jnp.zeros((KPAD, 8), F32).at[:k, :5].set(rois)

    e_p = pl.pallas_call(
        _main_body,
        grid=(NCHUNK // 4, 4),
        in_specs=[
            pl.BlockSpec((B, 8), lambda i, j: (i * 4 + j, 0)),
            pl.BlockSpec((100, 50 * C), lambda i, j: (0, 0)),
            pl.BlockSpec((16384, C), lambda i, j: (0, 0)),
            pl.BlockSpec((8, C), lambda i, j: (0, 0)),
        ],
        out_specs=pl.BlockSpec((4 * B, C), lambda i, j: (i, 0)),
        out_shape=jax.ShapeDtypeStruct((KPAD, C), F32),
        scratch_shapes=[pltpu.VMEM((4 * B, 16384), F32)],
        compiler_params=pltpu.CompilerParams(
            dimension_semantics=("arbitrary", "arbitrary")),
    )(rois_pad, fcat, wt, beff)

    cls_p, box_p = pl.pallas_call(
        _head_body,
        out_shape=[jax.ShapeDtypeStruct((KPAD, 81), F32),
                   jax.ShapeDtypeStruct((KPAD, 324), F32)],
    )(e_p, Wc1, _row8(bc1), Wc2, _row8(bc2),
      Wr1, _row8(br1), Wr2, _row8(br2),
      Wcls, _row8(bcls), Wbox, _row8(bbox))

    return (cls_p[:k], box_p[:k], e_p[:k].reshape(k, C, 1, 1))
